# Initial kernel scaffold; baseline (speedup 1.0000x reference)
import jax
import jax.numpy as jnp
from jax.experimental import pallas as pl


def kernel(scores, label, seqlen):
    raise NotImplementedError("write your pallas kernel here")


# speedup vs baseline: 5.3513x; 5.3513x over previous
import functools

import jax
import jax.numpy as jnp
from jax import lax
from jax.experimental import pallas as pl
from jax.experimental.pallas import tpu as pltpu
from jax.experimental.pallas import tpu_sc as plsc

B = 128
N = 8192
L = 16
NC, NS = 2, 16
NW = NC * NS
RPW = B // NW
BLK = 8 * L
NBLK = N // BLK

LO0 = 0x35000000
HI0 = 0x3F800000
SEARCH_ITERS = 28


def _sc_body(scores_hbm, seqlen_hbm, out_hbm, row_v, seq_v, vl_v):
    wid = lax.axis_index("s") * NC + lax.axis_index("c")
    pltpu.sync_copy(seqlen_hbm, seq_v.at[pl.ds(0, B)])
    lanes = lax.iota(jnp.int32, L)
    zeros_f = jnp.zeros((L,), jnp.float32)
    zeros_i = jnp.zeros((L,), jnp.int32)
    ones_i = jnp.ones((L,), jnp.int32)
    vl_vec = zeros_f

    for i in range(RPW):
        row = wid * RPW + i
        pltpu.sync_copy(scores_hbm.at[row], row_v)
        s = seq_v[pl.ds(row, L)][0]
        s_vec = jnp.full((L,), s, jnp.int32)
        k = (s >> 4) + 1
        nblocks = (s + (BLK - 1)) >> 7

        def mask_body(jc, _):
            pos = lanes + jc * L
            d = row_v[pl.ds(jc * L, L)]
            row_v[pl.ds(jc * L, L)] = jnp.where(pos < s_vec, d, zeros_f)
            return 0
        lax.fori_loop(s >> 4, nblocks * 8, mask_body, 0)

        def search_body(it, carry):
            lo, hi = carry
            mid = (lo + hi) >> 1
            t_vec = plsc.bitcast(jnp.full((L,), mid, jnp.int32), jnp.float32)

            def cnt_body(jb, acc):
                base = jb * BLK
                for u in range(8):
                    d = row_v[pl.ds(base + u * L, L)]
                    acc = acc + jnp.where(d >= t_vec, ones_i, zeros_i)
                return acc

            cnt = jnp.sum(lax.fori_loop(0, nblocks, cnt_body, zeros_i))
            ge = cnt >= k
            return jnp.where(ge, mid, lo), jnp.where(ge, hi, mid)

        lo, hi = lax.fori_loop(
            0, SEARCH_ITERS, search_body,
            (jnp.int32(LO0), jnp.int32(HI0)))

        t_vec = plsc.bitcast(jnp.full((L,), lo, jnp.int32), jnp.float32)

        def fin_body(jb, carry):
            sacc, cacc = carry
            base = jb * BLK
            for u in range(8):
                d = row_v[pl.ds(base + u * L, L)]
                gt = d > t_vec
                sacc = sacc + jnp.where(gt, d, zeros_f)
                cacc = cacc + jnp.where(gt, ones_i, zeros_i)
            return sacc, cacc

        sacc, cacc = lax.fori_loop(0, nblocks, fin_body, (zeros_f, zeros_i))
        sum_gt = jnp.sum(sacc)
        cnt_gt = jnp.sum(cacc)
        k_vec = jnp.full((L,), k, jnp.int32)
        tot_vec = jnp.full((L,), sum_gt) + (
            k_vec - jnp.full((L,), cnt_gt, jnp.int32)).astype(jnp.float32) * t_vec
        vl_vec = jnp.where(lanes == i, tot_vec, vl_vec)
        vl_vec = jnp.where(lanes == RPW + i, k_vec.astype(jnp.float32), vl_vec)

    vl_v[...] = vl_vec
    pltpu.sync_copy(vl_v, out_hbm.at[wid])


_sc_topk = pl.kernel(
    _sc_body,
    out_type=jax.ShapeDtypeStruct((NW, L), jnp.float32),
    mesh=plsc.VectorSubcoreMesh(core_axis_name="c", subcore_axis_name="s"),
    scratch_types=[
        pltpu.VMEM((N,), jnp.float32),
        pltpu.VMEM((B + L,), jnp.int32),
        pltpu.VMEM((L,), jnp.float32),
    ],
    compiler_params=pltpu.CompilerParams(needs_layout_passes=False),
)


def _tc_bce_body(vl_ref, lab_ref, out_ref):
    raw = vl_ref[...]
    v = raw[:, :RPW] / raw[:, RPW:2 * RPW]
    lab = lab_ref[...]
    terms = lab * jnp.log(v) + (1.0 - lab) * jnp.log(1.0 - v)
    out_ref[0, 0] = -jnp.sum(terms) / B


_tc_bce = pl.pallas_call(
    _tc_bce_body,
    out_shape=jax.ShapeDtypeStruct((1, 1), jnp.float32),
    out_specs=pl.BlockSpec(memory_space=pltpu.SMEM),
)


@jax.jit
def kernel(scores, label, seqlen):
    vl_raw = _sc_topk(scores, seqlen)
    loss = _tc_bce(vl_raw, label.reshape(NW, RPW))
    return loss[0, 0]
